# Initial kernel scaffold; baseline (speedup 1.0000x reference)
import jax
import jax.numpy as jnp
from jax.experimental import pallas as pl


def kernel(node_emb, W, b, gamma, beta, rel_emb, edge_index, e1, rel):
    raise NotImplementedError("write your pallas kernel here")


# speedup vs baseline: 11.0064x; 11.0064x over previous
import functools

import jax
import jax.numpy as jnp
from jax import lax
from jax.experimental import pallas as pl
from jax.experimental.pallas import tpu as pltpu
from jax.experimental.pallas import tpu_sc as plsc

NC = 2
NS = 16
NW = NC * NS
LANES = 16
BLK = 1024


def _sc_mesh():
    return plsc.VectorSubcoreMesh(
        core_axis_name="c", subcore_axis_name="s",
        num_cores=NC, num_subcores=NS)


def _make_deg_kernel(E, Np, B, R, D):
    Ew = E // NW
    CH = 80
    NCHUNK = Ew // CH
    BW = B // NW
    SL = Np // NS

    @functools.partial(
        pl.kernel,
        out_type=(
            jax.ShapeDtypeStruct((NC, Np), jnp.float32),
            jax.ShapeDtypeStruct((NC, Np), jnp.float32),
            jax.ShapeDtypeStruct((B, D), jnp.float32),
        ),
        mesh=_sc_mesh(),
        scratch_types=[
            pltpu.VMEM((CH,), jnp.int32),
            pltpu.VMEM((CH,), jnp.int32),
            pltpu.VMEM((CH,), jnp.float32),
            pltpu.VMEM((SL,), jnp.float32),
            pltpu.VMEM((BW,), jnp.int32),
            pltpu.VMEM((BW, D), jnp.float32),
            pltpu.VMEM_SHARED((Np,), jnp.float32),
            pltpu.VMEM_SHARED((Np,), jnp.float32),
            pltpu.SemaphoreType.DMA,
        ],
    )
    def k(src_h, dst_h, rel_h, rel_emb_h, dego_o, degi_o, embrel_o,
          sidx, didx, ones_v, zer_v, ridx, rrows, dego_sp, degi_sp, sem):
        c = lax.axis_index("c")
        s = lax.axis_index("s")
        w = s * NC + c
        for t in range(CH // LANES):
            ones_v[pl.ds(t * LANES, LANES)] = jnp.full((LANES,), 1.0,
                                                       jnp.float32)
        for t in range(SL // LANES):
            zer_v[pl.ds(t * LANES, LANES)] = jnp.zeros((LANES,), jnp.float32)
        pltpu.sync_copy(rel_h.at[pl.ds(w * BW, BW)], ridx)
        pltpu.async_copy(rel_emb_h.at[ridx], rrows, sem).wait()
        pltpu.sync_copy(rrows, embrel_o.at[pl.ds(w * BW, BW)])
        off0 = pl.multiple_of(s * SL, 8)
        pltpu.sync_copy(zer_v, dego_sp.at[pl.ds(off0, SL)])
        pltpu.sync_copy(zer_v, degi_sp.at[pl.ds(off0, SL)])
        plsc.subcore_barrier()

        def body(i, carry):
            base = pl.multiple_of(w * Ew + i * CH, 8)
            pltpu.sync_copy(src_h.at[pl.ds(base, CH)], sidx)
            pltpu.sync_copy(dst_h.at[pl.ds(base, CH)], didx)
            pltpu.sync_copy(ones_v, dego_sp.at[sidx], add=True)
            pltpu.sync_copy(ones_v, degi_sp.at[didx], add=True)
            return carry

        lax.fori_loop(0, NCHUNK, body, 0)
        plsc.subcore_barrier()
        pltpu.sync_copy(dego_sp.at[pl.ds(off0, SL)],
                        dego_o.at[c, pl.ds(off0, SL)])
        pltpu.sync_copy(degi_sp.at[pl.ds(off0, SL)],
                        degi_o.at[c, pl.ds(off0, SL)])

    return k


def _make_scale_kernel(Np, D):
    nb = Np // BLK

    def body(x_ref, dop_ref, dip_ref, y_ref, rin_ref):
        do = dop_ref[0] + dop_ref[1]
        di = dip_ref[0] + dip_ref[1]
        ro = lax.rsqrt(jnp.maximum(do, 1.0))
        y_ref[...] = x_ref[...] * ro
        rin_ref[...] = lax.rsqrt(jnp.maximum(di, 1.0))

    return pl.pallas_call(
        body,
        grid=(nb,),
        in_specs=[
            pl.BlockSpec((BLK, D), lambda j: (j, 0)),
            pl.BlockSpec((NC, BLK, 1), lambda j: (0, j, 0)),
            pl.BlockSpec((NC, BLK, 1), lambda j: (0, j, 0)),
        ],
        out_specs=[
            pl.BlockSpec((BLK, D), lambda j: (j, 0)),
            pl.BlockSpec((BLK, 1), lambda j: (j, 0)),
        ],
        out_shape=[
            jax.ShapeDtypeStruct((Np, D), jnp.float32),
            jax.ShapeDtypeStruct((Np, 1), jnp.float32),
        ],
    )


def _make_agg_kernel(E, Np, B, D):
    Ew = E // NW
    CH = 128
    NCHUNK = Ew // CH
    TAIL = Ew - NCHUNK * CH
    SL = Np // NS
    BW = B // NW
    BS = B // NS

    @functools.partial(
        pl.kernel,
        out_type=(
            jax.ShapeDtypeStruct((NC, Np, D), jnp.float32),
            jax.ShapeDtypeStruct((NC, B, D), jnp.float32),
            jax.ShapeDtypeStruct((B,), jnp.float32),
        ),
        mesh=_sc_mesh(),
        scratch_types=[
            pltpu.VMEM((LANES, D), jnp.float32),
            pltpu.VMEM((CH,), jnp.int32),
            pltpu.VMEM((CH,), jnp.int32),
            pltpu.VMEM((CH, D), jnp.float32),
            pltpu.VMEM((16,), jnp.int32),
            pltpu.VMEM((16,), jnp.int32),
            pltpu.VMEM((16, D), jnp.float32),
            pltpu.VMEM((BW,), jnp.int32),
            pltpu.VMEM((BW,), jnp.float32),
            pltpu.VMEM((BS,), jnp.int32),
            pltpu.VMEM((BS, D), jnp.float32),
            pltpu.VMEM_SHARED((Np, D), jnp.float32),
            pltpu.SemaphoreType.DMA,
        ],
    )
    def k(y_h, src_h, dst_h, e1_h, rin_h, agg_o, aggsel_o, rinsel_o,
          zblk, sidx, didx, rows, sidxt, didxt, rowst, eidx, rsel,
          eidx2, selrows, agg_sp, sem):
        c = lax.axis_index("c")
        s = lax.axis_index("s")
        w = s * NC + c
        for r in range(LANES):
            for t in range(D // LANES):
                zblk[r, pl.ds(t * LANES, LANES)] = jnp.zeros((LANES,),
                                                             jnp.float32)
        for t in range(SL // LANES):
            off = pl.multiple_of(s * SL + t * LANES, 8)
            pltpu.sync_copy(zblk, agg_sp.at[pl.ds(off, LANES), :])
        pltpu.sync_copy(e1_h.at[pl.ds(w * BW, BW)], eidx)
        pltpu.async_copy(rin_h.at[eidx], rsel, sem).wait()
        pltpu.sync_copy(rsel, rinsel_o.at[pl.ds(w * BW, BW)])
        plsc.subcore_barrier()

        def body(i, carry):
            base = pl.multiple_of(w * Ew + i * CH, 8)
            pltpu.sync_copy(src_h.at[pl.ds(base, CH)], sidx)
            pltpu.sync_copy(dst_h.at[pl.ds(base, CH)], didx)
            pltpu.async_copy(y_h.at[sidx], rows, sem).wait()
            pltpu.sync_copy(rows, agg_sp.at[didx], add=True)
            return carry

        lax.fori_loop(0, NCHUNK, body, 0)
        if TAIL:
            base = pl.multiple_of(w * Ew + NCHUNK * CH, 8)
            pltpu.sync_copy(src_h.at[pl.ds(base, TAIL)], sidxt)
            pltpu.sync_copy(dst_h.at[pl.ds(base, TAIL)], didxt)
            pltpu.async_copy(y_h.at[sidxt], rowst, sem).wait()
            pltpu.sync_copy(rowst, agg_sp.at[didxt], add=True)
        plsc.subcore_barrier()
        off0 = pl.multiple_of(s * SL, 8)
        pltpu.sync_copy(agg_sp.at[pl.ds(off0, SL), :],
                        agg_o.at[c, pl.ds(off0, SL), :])
        plsc.subcore_barrier()
        pltpu.sync_copy(e1_h.at[pl.ds(s * BS, BS)], eidx2)
        pltpu.async_copy(agg_o.at[c].at[eidx2], selrows, sem).wait()
        pltpu.sync_copy(selrows, aggsel_o.at[c, pl.ds(s * BS, BS), :])

    return k


def _make_head_kernel(Np, D, B, N):
    nb = Np // BLK

    def body(agg_ref, rin_ref, w_ref, b_ref, g_ref, be_ref,
             aggsel_ref, rinsel_ref, embrel_ref, out_ref,
             h_scr, s_acc, ss_acc, mean_s, rstd_s, q_scr):
        p = pl.program_id(0)
        j = pl.program_id(1)

        @pl.when(p == 0)
        def _pass0():
            agg = agg_ref[0] + agg_ref[1]
            a = agg * rin_ref[...]
            h = jnp.dot(a, w_ref[...],
                        preferred_element_type=jnp.float32) + b_ref[...]
            rowid = lax.broadcasted_iota(jnp.int32, (BLK, 1), 0) + j * BLK
            m = (rowid < N).astype(jnp.float32)
            hm = h * m

            @pl.when(j == 0)
            def _init():
                s_acc[...] = jnp.zeros_like(s_acc)
                ss_acc[...] = jnp.zeros_like(ss_acc)

            s_acc[...] += jnp.sum(hm, axis=0, keepdims=True)
            ss_acc[...] += jnp.sum(hm * hm, axis=0, keepdims=True)
            h_scr[pl.ds(j * BLK, BLK), :] = h

        @pl.when(p == 1)
        def _pass1():
            @pl.when(j == 0)
            def _head():
                mean = s_acc[...] * (1.0 / N)
                var = ss_acc[...] * (1.0 / N) - mean * mean
                rstd = lax.rsqrt(var + 1e-5)
                mean_s[...] = mean
                rstd_s[...] = rstd
                aggsel = aggsel_ref[0] + aggsel_ref[1]
                hsel = jnp.dot(aggsel * rinsel_ref[...], w_ref[...],
                               preferred_element_type=jnp.float32) + b_ref[...]
                embe1 = jnp.tanh((hsel - mean) * rstd * g_ref[...]
                                 + be_ref[...])
                q_scr[...] = embe1 * embrel_ref[...]

            hj = h_scr[pl.ds(j * BLK, BLK), :]
            nbk = jnp.tanh((hj - mean_s[...]) * rstd_s[...] * g_ref[...]
                           + be_ref[...])
            lg = lax.dot_general(q_scr[...], nbk,
                                 (((1,), (1,)), ((), ())),
                                 preferred_element_type=jnp.float32)
            out_ref[...] = jax.nn.sigmoid(lg)

    return pl.pallas_call(
        body,
        grid=(2, nb),
        in_specs=[
            pl.BlockSpec((NC, BLK, D), lambda p, j: (0, j * (1 - p), 0)),
            pl.BlockSpec((BLK, 1), lambda p, j: (j * (1 - p), 0)),
            pl.BlockSpec((D, D), lambda p, j: (0, 0)),
            pl.BlockSpec((1, D), lambda p, j: (0, 0)),
            pl.BlockSpec((1, D), lambda p, j: (0, 0)),
            pl.BlockSpec((1, D), lambda p, j: (0, 0)),
            pl.BlockSpec((NC, B, D), lambda p, j: (0, 0, 0)),
            pl.BlockSpec((B, 1), lambda p, j: (0, 0)),
            pl.BlockSpec((B, D), lambda p, j: (0, 0)),
        ],
        out_specs=pl.BlockSpec((B, BLK), lambda p, j: (0, p * j)),
        out_shape=jax.ShapeDtypeStruct((B, Np), jnp.float32),
        scratch_shapes=[
            pltpu.VMEM((Np, D), jnp.float32),
            pltpu.VMEM((1, D), jnp.float32),
            pltpu.VMEM((1, D), jnp.float32),
            pltpu.VMEM((1, D), jnp.float32),
            pltpu.VMEM((1, D), jnp.float32),
            pltpu.VMEM((B, D), jnp.float32),
        ],
    )


def kernel(node_emb, W, b, gamma, beta, rel_emb, edge_index, e1, rel):
    N, D = node_emb.shape
    E = edge_index.shape[1]
    B = e1.shape[0]
    R = rel_emb.shape[0]
    Np = ((N + BLK - 1) // BLK) * BLK

    src = edge_index[0]
    dst = edge_index[1]
    x_pad = jnp.pad(node_emb, ((0, Np - N), (0, 0)))

    dego_p, degi_p, emb_rel_sel = _make_deg_kernel(E, Np, B, R, D)(
        src, dst, rel, rel_emb)

    y, rin_col = _make_scale_kernel(Np, D)(
        x_pad, dego_p.reshape(NC, Np, 1), degi_p.reshape(NC, Np, 1))

    aggp, aggselp, rinsel = _make_agg_kernel(E, Np, B, D)(
        y, src, dst, e1, rin_col.reshape(Np))

    logits_pad = _make_head_kernel(Np, D, B, N)(
        aggp, rin_col, W, b.reshape(1, D), gamma.reshape(1, D),
        beta.reshape(1, D), aggselp, rinsel.reshape(B, 1), emb_rel_sel)

    return logits_pad[:, :N]


# speedup vs baseline: 21.1521x; 1.9218x over previous
import functools

import jax
import jax.numpy as jnp
from jax import lax
from jax.experimental import pallas as pl
from jax.experimental.pallas import tpu as pltpu
from jax.experimental.pallas import tpu_sc as plsc

NC = 2
NS = 16
NW = NC * NS
LANES = 16
BLK = 1024


def _sc_mesh():
    return plsc.VectorSubcoreMesh(
        core_axis_name="c", subcore_axis_name="s",
        num_cores=NC, num_subcores=NS)


def _make_deg_kernel(E, Np, B, R, D):
    CH = 125
    Ew = E // NW
    NCHUNK = Ew // CH
    NG = 4
    G = NCHUNK // NG
    assert NCHUNK == NG * G
    UNR = 4
    BW = B // NW
    SL = Np // NS

    @functools.partial(
        pl.kernel,
        out_type=(
            jax.ShapeDtypeStruct((NC, Np), jnp.float32),
            jax.ShapeDtypeStruct((NC, Np), jnp.float32),
            jax.ShapeDtypeStruct((B, D), jnp.float32),
        ),
        mesh=_sc_mesh(),
        scratch_types=[
            pltpu.VMEM((NG, G, CH), jnp.int32),
            pltpu.VMEM((NG, G, CH), jnp.int32),
            pltpu.VMEM((128,), jnp.float32),
            pltpu.VMEM((SL,), jnp.float32),
            pltpu.VMEM((BW,), jnp.int32),
            pltpu.VMEM((BW, D), jnp.float32),
            pltpu.VMEM_SHARED((Np,), jnp.float32),
            pltpu.VMEM_SHARED((Np,), jnp.float32),
            pltpu.SemaphoreType.DMA,
        ],
    )
    def k(e3_h, rel_h, rel_emb_h, dego_o, degi_o, embrel_o,
          sidx2, didx2, ones_v, zer_v, ridx, rrows, dego_sp, degi_sp, sem):
        c = lax.axis_index("c")
        s = lax.axis_index("s")
        w = s * NC + c
        for t in range(128 // LANES):
            ones_v[pl.ds(t * LANES, LANES)] = jnp.full((LANES,), 1.0,
                                                       jnp.float32)
        ones = ones_v.at[pl.ds(0, CH)]
        for t in range(SL // LANES):
            zer_v[pl.ds(t * LANES, LANES)] = jnp.zeros((LANES,), jnp.float32)
        pltpu.sync_copy(rel_h.at[pl.ds(w * BW, BW)], ridx)
        pltpu.async_copy(rel_emb_h.at[ridx], rrows, sem).wait()
        pltpu.sync_copy(rrows, embrel_o.at[pl.ds(w * BW, BW)])
        pltpu.sync_copy(e3_h.at[0, w], sidx2)
        pltpu.sync_copy(e3_h.at[1, w], didx2)
        off0 = pl.multiple_of(s * SL, 8)
        pltpu.sync_copy(zer_v, dego_sp.at[pl.ds(off0, SL)])
        pltpu.sync_copy(zer_v, degi_sp.at[pl.ds(off0, SL)])
        plsc.subcore_barrier()

        def body(p, carry):
            ds_ = []
            for u in range(UNR):
                i = p * UNR + u
                gi = i // G
                ri = i % G
                ds_.append(pltpu.async_copy(
                    ones, dego_sp.at[sidx2.at[gi, ri]], sem, add=True))
                ds_.append(pltpu.async_copy(
                    ones, degi_sp.at[didx2.at[gi, ri]], sem, add=True))
            for dsc in ds_:
                dsc.wait()
            return carry

        assert NCHUNK % UNR == 0
        lax.fori_loop(0, NCHUNK // UNR, body, 0)
        plsc.subcore_barrier()
        pltpu.sync_copy(dego_sp.at[pl.ds(off0, SL)],
                        dego_o.at[c, pl.ds(off0, SL)])
        pltpu.sync_copy(degi_sp.at[pl.ds(off0, SL)],
                        degi_o.at[c, pl.ds(off0, SL)])

    return k


def _make_scale_kernel(N, Np, D):
    nb = Np // BLK

    def body(x_ref, dop_ref, dip_ref, y_ref, rin_ref):
        do = dop_ref[0] + dop_ref[1]
        di = dip_ref[0] + dip_ref[1]
        ro = lax.rsqrt(jnp.maximum(do, 1.0))
        y_ref[...] = x_ref[...] * ro
        rin_ref[...] = lax.rsqrt(jnp.maximum(di, 1.0))

    return pl.pallas_call(
        body,
        grid=(nb,),
        in_specs=[
            pl.BlockSpec((BLK, D), lambda j: (j, 0)),
            pl.BlockSpec((NC, BLK, 1), lambda j: (0, j, 0)),
            pl.BlockSpec((NC, BLK, 1), lambda j: (0, j, 0)),
        ],
        out_specs=[
            pl.BlockSpec((BLK, D), lambda j: (j, 0)),
            pl.BlockSpec((BLK, 1), lambda j: (j, 0)),
        ],
        out_shape=[
            jax.ShapeDtypeStruct((N, D), jnp.float32),
            jax.ShapeDtypeStruct((Np, 1), jnp.float32),
        ],
    )


def _make_agg_kernel(E, Np, B, D):
    CH = 125
    Ew = E // NW
    NCHUNK = Ew // CH
    NG = 4
    G = NCHUNK // NG
    assert NCHUNK == NG * G and G % 2 == 0
    SL = Np // NS
    BW = B // NW
    BS = B // NS

    @functools.partial(
        pl.kernel,
        out_type=(
            jax.ShapeDtypeStruct((NC, Np, D), jnp.float32),
            jax.ShapeDtypeStruct((NC, B, D), jnp.float32),
            jax.ShapeDtypeStruct((B,), jnp.float32),
        ),
        mesh=_sc_mesh(),
        scratch_types=[
            pltpu.VMEM((LANES, D), jnp.float32),
            pltpu.VMEM((G, CH), jnp.int32),
            pltpu.VMEM((G, CH), jnp.int32),
            pltpu.VMEM((CH, D), jnp.float32),
            pltpu.VMEM((CH, D), jnp.float32),
            pltpu.VMEM((BW,), jnp.int32),
            pltpu.VMEM((BW,), jnp.float32),
            pltpu.VMEM((BS,), jnp.int32),
            pltpu.VMEM((BS // 2, D), jnp.float32),
            pltpu.VMEM_SHARED((Np, D), jnp.float32),
            pltpu.SemaphoreType.DMA,
            pltpu.SemaphoreType.DMA,
            pltpu.SemaphoreType.DMA,
        ],
    )
    def k(y_h, e3_h, e1_h, rin_h, agg_o, aggsel_o, rinsel_o,
          zblk, sidx2, didx2, rowsA, rowsB, eidx, rsel,
          eidx2, selrows, agg_sp, gsA, gsB, sem):
        c = lax.axis_index("c")
        s = lax.axis_index("s")
        w = s * NC + c
        for r in range(LANES):
            for t in range(D // LANES):
                zblk[r, pl.ds(t * LANES, LANES)] = jnp.zeros((LANES,),
                                                             jnp.float32)
        for t in range(SL // LANES):
            off = pl.multiple_of(s * SL + t * LANES, 8)
            pltpu.sync_copy(zblk, agg_sp.at[pl.ds(off, LANES), :])
        pltpu.sync_copy(e1_h.at[pl.ds(w * BW, BW)], eidx)
        pltpu.async_copy(rin_h.at[eidx], rsel, sem).wait()
        pltpu.sync_copy(rsel, rinsel_o.at[pl.ds(w * BW, BW)])
        plsc.subcore_barrier()

        for g in range(NG):
            pltpu.sync_copy(e3_h.at[0, w, g], sidx2)
            pltpu.sync_copy(e3_h.at[1, w, g], didx2)
            pltpu.async_copy(y_h.at[sidx2.at[0]], rowsA, gsA)

            def body(p, carry):
                i0 = 2 * p
                pltpu.async_copy(y_h.at[sidx2.at[i0 + 1]], rowsB, gsB)
                pltpu.make_async_copy(y_h.at[sidx2.at[i0]],
                                      rowsA, gsA).wait()
                pltpu.sync_copy(rowsA, agg_sp.at[didx2.at[i0]], add=True)

                @pl.when(i0 + 2 < G)
                def _prefetch():
                    pltpu.async_copy(y_h.at[sidx2.at[i0 + 2]], rowsA, gsA)

                pltpu.make_async_copy(y_h.at[sidx2.at[i0 + 1]],
                                      rowsB, gsB).wait()
                pltpu.sync_copy(rowsB, agg_sp.at[didx2.at[i0 + 1]], add=True)
                return carry

            lax.fori_loop(0, G // 2, body, 0)
        plsc.subcore_barrier()
        off0 = pl.multiple_of(s * SL, 8)
        pltpu.sync_copy(agg_sp.at[pl.ds(off0, SL), :],
                        agg_o.at[c, pl.ds(off0, SL), :])
        plsc.subcore_barrier()
        pltpu.sync_copy(e1_h.at[pl.ds(s * BS, BS)], eidx2)
        H = BS // 2
        for hh in range(2):
            pltpu.async_copy(agg_o.at[c].at[eidx2.at[pl.ds(hh * H, H)]],
                             selrows, sem).wait()
            pltpu.sync_copy(selrows,
                            aggsel_o.at[c, pl.ds(s * BS + hh * H, H), :])

    return k


def _make_head_kernel(Np, D, B, N):
    nb = Np // BLK

    def body(agg_ref, rin_ref, w_ref, b_ref, g_ref, be_ref,
             aggsel_ref, rinsel_ref, embrel_ref, out_ref,
             h_scr, s_acc, ss_acc, mean_s, rstd_s, q_scr):
        p = pl.program_id(0)
        j = pl.program_id(1)

        @pl.when(p == 0)
        def _pass0():
            agg = agg_ref[0] + agg_ref[1]
            a = agg * rin_ref[...]
            h = jnp.dot(a, w_ref[...],
                        preferred_element_type=jnp.float32) + b_ref[...]
            rowid = lax.broadcasted_iota(jnp.int32, (BLK, 1), 0) + j * BLK
            hm = jnp.where(rowid < N, h, 0.0)

            @pl.when(j == 0)
            def _init():
                s_acc[...] = jnp.zeros_like(s_acc)
                ss_acc[...] = jnp.zeros_like(ss_acc)

            s_acc[...] += jnp.sum(hm, axis=0, keepdims=True)
            ss_acc[...] += jnp.sum(hm * hm, axis=0, keepdims=True)
            h_scr[pl.ds(j * BLK, BLK), :] = h

        @pl.when(p == 1)
        def _pass1():
            @pl.when(j == 0)
            def _head():
                mean = s_acc[...] * (1.0 / N)
                var = ss_acc[...] * (1.0 / N) - mean * mean
                rstd = lax.rsqrt(var + 1e-5)
                mean_s[...] = mean
                rstd_s[...] = rstd
                aggsel = aggsel_ref[0] + aggsel_ref[1]
                hsel = jnp.dot(aggsel * rinsel_ref[...], w_ref[...],
                               preferred_element_type=jnp.float32) + b_ref[...]
                embe1 = jnp.tanh((hsel - mean) * rstd * g_ref[...]
                                 + be_ref[...])
                q_scr[...] = embe1 * embrel_ref[...]

            hj = h_scr[pl.ds(j * BLK, BLK), :]
            nbk = jnp.tanh((hj - mean_s[...]) * rstd_s[...] * g_ref[...]
                           + be_ref[...])
            lg = lax.dot_general(q_scr[...], nbk,
                                 (((1,), (1,)), ((), ())),
                                 preferred_element_type=jnp.float32)
            out_ref[...] = jax.nn.sigmoid(lg)

    return pl.pallas_call(
        body,
        grid=(2, nb),
        in_specs=[
            pl.BlockSpec((NC, BLK, D), lambda p, j: (0, j * (1 - p), 0)),
            pl.BlockSpec((BLK, 1), lambda p, j: (j * (1 - p), 0)),
            pl.BlockSpec((D, D), lambda p, j: (0, 0)),
            pl.BlockSpec((1, D), lambda p, j: (0, 0)),
            pl.BlockSpec((1, D), lambda p, j: (0, 0)),
            pl.BlockSpec((1, D), lambda p, j: (0, 0)),
            pl.BlockSpec((NC, B, D), lambda p, j: (0, 0, 0)),
            pl.BlockSpec((B, 1), lambda p, j: (0, 0)),
            pl.BlockSpec((B, D), lambda p, j: (0, 0)),
        ],
        out_specs=pl.BlockSpec((B, BLK), lambda p, j: (0, p * j)),
        out_shape=jax.ShapeDtypeStruct((B, Np), jnp.float32),
        scratch_shapes=[
            pltpu.VMEM((Np, D), jnp.float32),
            pltpu.VMEM((1, D), jnp.float32),
            pltpu.VMEM((1, D), jnp.float32),
            pltpu.VMEM((1, D), jnp.float32),
            pltpu.VMEM((1, D), jnp.float32),
            pltpu.VMEM((B, D), jnp.float32),
        ],
    )


def kernel(node_emb, W, b, gamma, beta, rel_emb, edge_index, e1, rel):
    N, D = node_emb.shape
    E = edge_index.shape[1]
    B = e1.shape[0]
    R = rel_emb.shape[0]
    Np = ((N + BLK - 1) // BLK) * BLK

    e3 = edge_index.reshape(2, NW, 4, E // (NW * 4 * 125), 125)

    dego_p, degi_p, emb_rel_sel = _make_deg_kernel(E, Np, B, R, D)(
        e3, rel, rel_emb)

    y, rin_col = _make_scale_kernel(N, Np, D)(
        node_emb, dego_p.reshape(NC, Np, 1), degi_p.reshape(NC, Np, 1))

    aggp, aggselp, rinsel = _make_agg_kernel(E, Np, B, D)(
        y, e3, e1, rin_col.reshape(Np))

    logits_pad = _make_head_kernel(Np, D, B, N)(
        aggp, rin_col, W, b.reshape(1, D), gamma.reshape(1, D),
        beta.reshape(1, D), aggselp, rinsel.reshape(B, 1), emb_rel_sel)

    return logits_pad[:, :N]


# speedup vs baseline: 22.9330x; 1.0842x over previous
import functools

import jax
import jax.numpy as jnp
from jax import lax
from jax.experimental import pallas as pl
from jax.experimental.pallas import tpu as pltpu
from jax.experimental.pallas import tpu_sc as plsc

NC = 2
NS = 16
NW = NC * NS
LANES = 16
BLK = 1024


def _sc_mesh():
    return plsc.VectorSubcoreMesh(
        core_axis_name="c", subcore_axis_name="s",
        num_cores=NC, num_subcores=NS)


def _make_deg_kernel(E, Np, B, R, D):
    CH = 125
    Ew = E // NW
    NCHUNK = Ew // CH
    NG = 4
    G = NCHUNK // NG
    assert NCHUNK == NG * G
    UNR = 4
    BW = B // NW
    SL = Np // NS

    @functools.partial(
        pl.kernel,
        out_type=(
            jax.ShapeDtypeStruct((NC, Np), jnp.float32),
            jax.ShapeDtypeStruct((NC, Np), jnp.float32),
            jax.ShapeDtypeStruct((B, D), jnp.float32),
        ),
        mesh=_sc_mesh(),
        scratch_types=[
            pltpu.VMEM((NG, G, CH), jnp.int32),
            pltpu.VMEM((NG, G, CH), jnp.int32),
            pltpu.VMEM((128,), jnp.float32),
            pltpu.VMEM((SL,), jnp.float32),
            pltpu.VMEM((BW,), jnp.int32),
            pltpu.VMEM((BW, D), jnp.float32),
            pltpu.VMEM_SHARED((Np,), jnp.float32),
            pltpu.VMEM_SHARED((Np,), jnp.float32),
            pltpu.SemaphoreType.DMA,
        ],
    )
    def k(e3_h, rel_h, rel_emb_h, dego_o, degi_o, embrel_o,
          sidx2, didx2, ones_v, zer_v, ridx, rrows, dego_sp, degi_sp, sem):
        c = lax.axis_index("c")
        s = lax.axis_index("s")
        w = s * NC + c
        for t in range(128 // LANES):
            ones_v[pl.ds(t * LANES, LANES)] = jnp.full((LANES,), 1.0,
                                                       jnp.float32)
        ones = ones_v.at[pl.ds(0, CH)]
        for t in range(SL // LANES):
            zer_v[pl.ds(t * LANES, LANES)] = jnp.zeros((LANES,), jnp.float32)
        pltpu.sync_copy(rel_h.at[pl.ds(w * BW, BW)], ridx)
        pltpu.async_copy(rel_emb_h.at[ridx], rrows, sem).wait()
        pltpu.sync_copy(rrows, embrel_o.at[pl.ds(w * BW, BW)])
        pltpu.sync_copy(e3_h.at[0, w], sidx2)
        pltpu.sync_copy(e3_h.at[1, w], didx2)
        off0 = pl.multiple_of(s * SL, 8)
        pltpu.sync_copy(zer_v, dego_sp.at[pl.ds(off0, SL)])
        pltpu.sync_copy(zer_v, degi_sp.at[pl.ds(off0, SL)])
        plsc.subcore_barrier()

        def body(p, carry):
            ds_ = []
            for u in range(UNR):
                i = p * UNR + u
                gi = i // G
                ri = i % G
                ds_.append(pltpu.async_copy(
                    ones, dego_sp.at[sidx2.at[gi, ri]], sem, add=True))
                ds_.append(pltpu.async_copy(
                    ones, degi_sp.at[didx2.at[gi, ri]], sem, add=True))
            for dsc in ds_:
                dsc.wait()
            return carry

        assert NCHUNK % UNR == 0
        lax.fori_loop(0, NCHUNK // UNR, body, 0)
        plsc.subcore_barrier()
        pltpu.sync_copy(dego_sp.at[pl.ds(off0, SL)],
                        dego_o.at[c, pl.ds(off0, SL)])
        pltpu.sync_copy(degi_sp.at[pl.ds(off0, SL)],
                        degi_o.at[c, pl.ds(off0, SL)])

    return k


def _make_scale_kernel(N, Np, D):
    nb = Np // BLK

    def body(x_ref, dop_ref, dip_ref, y_ref, rin_ref):
        do = dop_ref[0] + dop_ref[1]
        di = dip_ref[0] + dip_ref[1]
        ro = lax.rsqrt(jnp.maximum(do, 1.0))
        y_ref[...] = x_ref[...] * ro
        rin_ref[...] = lax.rsqrt(jnp.maximum(di, 1.0))

    return pl.pallas_call(
        body,
        grid=(nb,),
        in_specs=[
            pl.BlockSpec((BLK, D), lambda j: (j, 0)),
            pl.BlockSpec((NC, BLK, 1), lambda j: (0, j, 0)),
            pl.BlockSpec((NC, BLK, 1), lambda j: (0, j, 0)),
        ],
        out_specs=[
            pl.BlockSpec((BLK, D), lambda j: (j, 0)),
            pl.BlockSpec((BLK, 1), lambda j: (j, 0)),
        ],
        out_shape=[
            jax.ShapeDtypeStruct((N, D), jnp.float32),
            jax.ShapeDtypeStruct((Np, 1), jnp.float32),
        ],
    )


def _make_agg_kernel(E, Np, B, D):
    CH = 125
    Ew = E // NW
    NCHUNK = Ew // CH
    NG = 4
    G = NCHUNK // NG
    assert NCHUNK == NG * G and G % 2 == 0
    SL = Np // NS
    BW = B // NW
    BS = B // NS

    @functools.partial(
        pl.kernel,
        out_type=(
            jax.ShapeDtypeStruct((NC, Np, D), jnp.float32),
            jax.ShapeDtypeStruct((NC, B, D), jnp.float32),
            jax.ShapeDtypeStruct((B,), jnp.float32),
        ),
        mesh=_sc_mesh(),
        scratch_types=[
            pltpu.VMEM((LANES, D), jnp.float32),
            pltpu.VMEM((G, CH), jnp.int32),
            pltpu.VMEM((G, CH), jnp.int32),
            pltpu.VMEM((CH, D), jnp.float32),
            pltpu.VMEM((CH, D), jnp.float32),
            pltpu.VMEM((BW,), jnp.int32),
            pltpu.VMEM((BW,), jnp.float32),
            pltpu.VMEM((BS,), jnp.int32),
            pltpu.VMEM((BS // 2, D), jnp.float32),
            pltpu.VMEM_SHARED((Np, D), jnp.float32),
            pltpu.SemaphoreType.DMA,
            pltpu.SemaphoreType.DMA,
            pltpu.SemaphoreType.DMA,
        ],
    )
    def k(y_h, e3_h, e1_h, rin_h, agg_o, aggsel_o, rinsel_o,
          zblk, sidx2, didx2, rowsA, rowsB, eidx, rsel,
          eidx2, selrows, agg_sp, gsA, gsB, sem):
        c = lax.axis_index("c")
        s = lax.axis_index("s")
        w = s * NC + c
        for r in range(LANES):
            for t in range(D // LANES):
                zblk[r, pl.ds(t * LANES, LANES)] = jnp.zeros((LANES,),
                                                             jnp.float32)
        for t in range(SL // LANES):
            off = pl.multiple_of(s * SL + t * LANES, 8)
            pltpu.sync_copy(zblk, agg_sp.at[pl.ds(off, LANES), :])
        pltpu.sync_copy(e1_h.at[pl.ds(w * BW, BW)], eidx)
        pltpu.async_copy(rin_h.at[eidx], rsel, sem).wait()
        pltpu.sync_copy(rsel, rinsel_o.at[pl.ds(w * BW, BW)])
        plsc.subcore_barrier()

        for g in range(NG):
            pltpu.sync_copy(e3_h.at[0, w, g], sidx2)
            pltpu.sync_copy(e3_h.at[1, w, g], didx2)
            pltpu.async_copy(y_h.at[sidx2.at[0]], rowsA, gsA)

            def body(p, carry):
                i0 = 2 * p
                pltpu.async_copy(y_h.at[sidx2.at[i0 + 1]], rowsB, gsB)
                pltpu.make_async_copy(y_h.at[sidx2.at[i0]],
                                      rowsA, gsA).wait()
                pltpu.sync_copy(rowsA, agg_sp.at[didx2.at[i0]], add=True)

                @pl.when(i0 + 2 < G)
                def _prefetch():
                    pltpu.async_copy(y_h.at[sidx2.at[i0 + 2]], rowsA, gsA)

                pltpu.make_async_copy(y_h.at[sidx2.at[i0 + 1]],
                                      rowsB, gsB).wait()
                pltpu.sync_copy(rowsB, agg_sp.at[didx2.at[i0 + 1]], add=True)
                return carry

            lax.fori_loop(0, G // 2, body, 0)
        plsc.subcore_barrier()
        off0 = pl.multiple_of(s * SL, 8)
        pltpu.sync_copy(agg_sp.at[pl.ds(off0, SL), :],
                        agg_o.at[c, pl.ds(off0, SL), :])
        plsc.subcore_barrier()
        pltpu.sync_copy(e1_h.at[pl.ds(s * BS, BS)], eidx2)
        H = BS // 2
        for hh in range(2):
            pltpu.async_copy(agg_o.at[c].at[eidx2.at[pl.ds(hh * H, H)]],
                             selrows, sem).wait()
            pltpu.sync_copy(selrows,
                            aggsel_o.at[c, pl.ds(s * BS + hh * H, H), :])

    return k


def _make_head_kernel(Np, D, B, N):
    nb = Np // BLK

    def body(agg_ref, rin_ref, w_ref, b_ref, g_ref, be_ref,
             aggsel_ref, rinsel_ref, embrel_ref, out_ref,
             h_scr, s_acc, ss_acc, mean_s, rstd_s, q_scr):
        p = pl.program_id(0)
        j = pl.program_id(1)

        @pl.when(p == 0)
        def _pass0():
            agg = agg_ref[0] + agg_ref[1]
            a = agg * rin_ref[...]
            h = jnp.dot(a, w_ref[...],
                        preferred_element_type=jnp.float32) + b_ref[...]
            rowid = lax.broadcasted_iota(jnp.int32, (BLK, 1), 0) + j * BLK
            hm = jnp.where(rowid < N, h, 0.0)

            @pl.when(j == 0)
            def _init():
                s_acc[...] = jnp.zeros_like(s_acc)
                ss_acc[...] = jnp.zeros_like(ss_acc)

            s_acc[...] += jnp.sum(hm, axis=0, keepdims=True)
            ss_acc[...] += jnp.sum(hm * hm, axis=0, keepdims=True)
            h_scr[pl.ds(j * BLK, BLK), :] = h

        @pl.when(p == 1)
        def _pass1():
            @pl.when(j == 0)
            def _head():
                mean = s_acc[...] * (1.0 / N)
                var = ss_acc[...] * (1.0 / N) - mean * mean
                rstd = lax.rsqrt(var + 1e-5)
                mean_s[...] = mean
                rstd_s[...] = rstd
                aggsel = aggsel_ref[0] + aggsel_ref[1]
                hsel = jnp.dot(aggsel * rinsel_ref[...], w_ref[...],
                               preferred_element_type=jnp.float32) + b_ref[...]
                embe1 = jnp.tanh((hsel - mean) * rstd * g_ref[...]
                                 + be_ref[...])
                q_scr[...] = embe1 * embrel_ref[...]

            hj = h_scr[pl.ds(j * BLK, BLK), :]
            nbk = jnp.tanh((hj - mean_s[...]) * rstd_s[...] * g_ref[...]
                           + be_ref[...])
            lg = lax.dot_general(q_scr[...], nbk,
                                 (((1,), (1,)), ((), ())),
                                 preferred_element_type=jnp.float32)
            out_ref[...] = jax.nn.sigmoid(lg)

    return pl.pallas_call(
        body,
        grid=(2, nb),
        in_specs=[
            pl.BlockSpec((NC, BLK, D), lambda p, j: (0, j * (1 - p), 0)),
            pl.BlockSpec((BLK, 1), lambda p, j: (j * (1 - p), 0)),
            pl.BlockSpec((D, D), lambda p, j: (0, 0)),
            pl.BlockSpec((1, D), lambda p, j: (0, 0)),
            pl.BlockSpec((1, D), lambda p, j: (0, 0)),
            pl.BlockSpec((1, D), lambda p, j: (0, 0)),
            pl.BlockSpec((NC, B, D), lambda p, j: (0, 0, 0)),
            pl.BlockSpec((B, 1), lambda p, j: (0, 0)),
            pl.BlockSpec((B, D), lambda p, j: (0, 0)),
        ],
        out_specs=pl.BlockSpec((B, BLK), lambda p, j: (0, p * j)),
        out_shape=jax.ShapeDtypeStruct((B, N), jnp.float32),
        scratch_shapes=[
            pltpu.VMEM((Np, D), jnp.float32),
            pltpu.VMEM((1, D), jnp.float32),
            pltpu.VMEM((1, D), jnp.float32),
            pltpu.VMEM((1, D), jnp.float32),
            pltpu.VMEM((1, D), jnp.float32),
            pltpu.VMEM((B, D), jnp.float32),
        ],
    )


def kernel(node_emb, W, b, gamma, beta, rel_emb, edge_index, e1, rel):
    N, D = node_emb.shape
    E = edge_index.shape[1]
    B = e1.shape[0]
    R = rel_emb.shape[0]
    Np = ((N + BLK - 1) // BLK) * BLK

    e3 = edge_index.reshape(2, NW, 4, E // (NW * 4 * 125), 125)

    dego_p, degi_p, emb_rel_sel = _make_deg_kernel(E, Np, B, R, D)(
        e3, rel, rel_emb)

    y, rin_col = _make_scale_kernel(N, Np, D)(
        node_emb, dego_p.reshape(NC, Np, 1), degi_p.reshape(NC, Np, 1))

    aggp, aggselp, rinsel = _make_agg_kernel(E, Np, B, D)(
        y, e3, e1, rin_col.reshape(Np))

    return _make_head_kernel(Np, D, B, N)(
        aggp, rin_col, W, b.reshape(1, D), gamma.reshape(1, D),
        beta.reshape(1, D), aggselp, rinsel.reshape(B, 1), emb_rel_sel)


# speedup vs baseline: 22.9815x; 1.0021x over previous
import functools

import jax
import jax.numpy as jnp
from jax import lax
from jax.experimental import pallas as pl
from jax.experimental.pallas import tpu as pltpu
from jax.experimental.pallas import tpu_sc as plsc

NC = 2
NS = 16
NW = NC * NS
LANES = 16
BLK = 1024


def _sc_mesh():
    return plsc.VectorSubcoreMesh(
        core_axis_name="c", subcore_axis_name="s",
        num_cores=NC, num_subcores=NS)


def _make_deg_kernel(E, Np, B, R, D):
    CH = 125
    Ew = E // NW
    NCHUNK = Ew // CH
    NG = 4
    G = NCHUNK // NG
    assert NCHUNK == NG * G
    UNR = 4
    BW = B // NW
    SL = Np // NS

    @functools.partial(
        pl.kernel,
        out_type=(
            jax.ShapeDtypeStruct((NC, Np), jnp.float32),
            jax.ShapeDtypeStruct((NC, Np), jnp.float32),
            jax.ShapeDtypeStruct((B, D), jnp.float32),
        ),
        mesh=_sc_mesh(),
        scratch_types=[
            pltpu.VMEM((NG, G, CH), jnp.int32),
            pltpu.VMEM((NG, G, CH), jnp.int32),
            pltpu.VMEM((128,), jnp.float32),
            pltpu.VMEM((SL,), jnp.float32),
            pltpu.VMEM((BW,), jnp.int32),
            pltpu.VMEM((BW, D), jnp.float32),
            pltpu.VMEM_SHARED((Np,), jnp.float32),
            pltpu.VMEM_SHARED((Np,), jnp.float32),
            pltpu.SemaphoreType.DMA,
        ],
    )
    def k(e3_h, rel_h, rel_emb_h, dego_o, degi_o, embrel_o,
          sidx2, didx2, ones_v, zer_v, ridx, rrows, dego_sp, degi_sp, sem):
        c = lax.axis_index("c")
        s = lax.axis_index("s")
        w = s * NC + c
        for t in range(128 // LANES):
            ones_v[pl.ds(t * LANES, LANES)] = jnp.full((LANES,), 1.0,
                                                       jnp.float32)
        ones = ones_v.at[pl.ds(0, CH)]
        for t in range(SL // LANES):
            zer_v[pl.ds(t * LANES, LANES)] = jnp.zeros((LANES,), jnp.float32)
        pltpu.sync_copy(rel_h.at[pl.ds(w * BW, BW)], ridx)
        pltpu.async_copy(rel_emb_h.at[ridx], rrows, sem).wait()
        pltpu.sync_copy(rrows, embrel_o.at[pl.ds(w * BW, BW)])
        pltpu.sync_copy(e3_h.at[0, w], sidx2)
        pltpu.sync_copy(e3_h.at[1, w], didx2)
        off0 = pl.multiple_of(s * SL, 8)
        pltpu.sync_copy(zer_v, dego_sp.at[pl.ds(off0, SL)])
        pltpu.sync_copy(zer_v, degi_sp.at[pl.ds(off0, SL)])
        plsc.subcore_barrier()

        def body(p, carry):
            ds_ = []
            for u in range(UNR):
                i = p * UNR + u
                gi = i // G
                ri = i % G
                ds_.append(pltpu.async_copy(
                    ones, dego_sp.at[sidx2.at[gi, ri]], sem, add=True))
                ds_.append(pltpu.async_copy(
                    ones, degi_sp.at[didx2.at[gi, ri]], sem, add=True))
            for dsc in ds_:
                dsc.wait()
            return carry

        assert NCHUNK % UNR == 0
        lax.fori_loop(0, NCHUNK // UNR, body, 0)
        plsc.subcore_barrier()
        pltpu.sync_copy(dego_sp.at[pl.ds(off0, SL)],
                        dego_o.at[c, pl.ds(off0, SL)])
        pltpu.sync_copy(degi_sp.at[pl.ds(off0, SL)],
                        degi_o.at[c, pl.ds(off0, SL)])

    return k


def _make_scale_kernel(N, Np, D):
    nb = Np // BLK

    def body(x_ref, dop_ref, dip_ref, y_ref, rin_ref):
        do = dop_ref[0] + dop_ref[1]
        di = dip_ref[0] + dip_ref[1]
        ro = lax.rsqrt(jnp.maximum(do, 1.0))
        y_ref[...] = x_ref[...] * ro
        rin_ref[...] = lax.rsqrt(jnp.maximum(di, 1.0))

    return pl.pallas_call(
        body,
        grid=(nb,),
        in_specs=[
            pl.BlockSpec((BLK, D), lambda j: (j, 0)),
            pl.BlockSpec((NC, BLK, 1), lambda j: (0, j, 0)),
            pl.BlockSpec((NC, BLK, 1), lambda j: (0, j, 0)),
        ],
        out_specs=[
            pl.BlockSpec((BLK, D), lambda j: (j, 0)),
            pl.BlockSpec((BLK, 1), lambda j: (j, 0)),
        ],
        out_shape=[
            jax.ShapeDtypeStruct((N, D), jnp.float32),
            jax.ShapeDtypeStruct((Np, 1), jnp.float32),
        ],
    )


def _make_agg_kernel(E, Np, B, D):
    CH = 125
    Ew = E // NW
    NCHUNK = Ew // CH
    NG = 4
    G = NCHUNK // NG
    assert NCHUNK == NG * G and G % 2 == 0
    SL = Np // NS
    BW = B // NW
    BS = B // NS

    @functools.partial(
        pl.kernel,
        out_type=(
            jax.ShapeDtypeStruct((NC, Np, D), jnp.float32),
            jax.ShapeDtypeStruct((NC, B, D), jnp.float32),
            jax.ShapeDtypeStruct((B,), jnp.float32),
        ),
        mesh=_sc_mesh(),
        scratch_types=[
            pltpu.VMEM((LANES, D), jnp.float32),
            pltpu.VMEM((G, CH), jnp.int32),
            pltpu.VMEM((G, CH), jnp.int32),
            pltpu.VMEM((CH, D), jnp.float32),
            pltpu.VMEM((CH, D), jnp.float32),
            pltpu.VMEM((BW,), jnp.int32),
            pltpu.VMEM((BW,), jnp.float32),
            pltpu.VMEM((BS,), jnp.int32),
            pltpu.VMEM((BS // 2, D), jnp.float32),
            pltpu.VMEM_SHARED((Np, D), jnp.float32),
            pltpu.SemaphoreType.DMA,
            pltpu.SemaphoreType.DMA,
            pltpu.SemaphoreType.DMA,
        ],
    )
    def k(y_h, e3_h, e1_h, rin_h, agg_o, aggsel_o, rinsel_o,
          zblk, sidx2, didx2, rowsA, rowsB, eidx, rsel,
          eidx2, selrows, agg_sp, gsA, gsB, sem):
        c = lax.axis_index("c")
        s = lax.axis_index("s")
        w = s * NC + c
        for r in range(LANES):
            for t in range(D // LANES):
                zblk[r, pl.ds(t * LANES, LANES)] = jnp.zeros((LANES,),
                                                             jnp.float32)
        for t in range(SL // LANES):
            off = pl.multiple_of(s * SL + t * LANES, 8)
            pltpu.sync_copy(zblk, agg_sp.at[pl.ds(off, LANES), :])
        pltpu.sync_copy(e1_h.at[pl.ds(w * BW, BW)], eidx)
        pltpu.async_copy(rin_h.at[eidx], rsel, sem).wait()
        pltpu.sync_copy(rsel, rinsel_o.at[pl.ds(w * BW, BW)])
        plsc.subcore_barrier()

        for g in range(NG):
            pltpu.sync_copy(e3_h.at[0, w, g], sidx2)
            pltpu.sync_copy(e3_h.at[1, w, g], didx2)
            pltpu.async_copy(y_h.at[sidx2.at[0]], rowsA, gsA)

            def body(p, carry):
                i0 = 2 * p
                pltpu.async_copy(y_h.at[sidx2.at[i0 + 1]], rowsB, gsB)
                pltpu.make_async_copy(y_h.at[sidx2.at[i0]],
                                      rowsA, gsA).wait()
                pltpu.async_copy(rowsA, agg_sp.at[didx2.at[i0]],
                                 sem, add=True).wait()

                @pl.when(i0 + 2 < G)
                def _prefetch():
                    pltpu.async_copy(y_h.at[sidx2.at[i0 + 2]], rowsA, gsA)

                pltpu.make_async_copy(y_h.at[sidx2.at[i0 + 1]],
                                      rowsB, gsB).wait()
                pltpu.async_copy(rowsB, agg_sp.at[didx2.at[i0 + 1]],
                                 sem, add=True).wait()
                return carry

            lax.fori_loop(0, G // 2, body, 0)
        plsc.subcore_barrier()
        off0 = pl.multiple_of(s * SL, 8)
        pltpu.sync_copy(agg_sp.at[pl.ds(off0, SL), :],
                        agg_o.at[c, pl.ds(off0, SL), :])
        plsc.subcore_barrier()
        pltpu.sync_copy(e1_h.at[pl.ds(s * BS, BS)], eidx2)
        H = BS // 2
        for hh in range(2):
            pltpu.async_copy(agg_o.at[c].at[eidx2.at[pl.ds(hh * H, H)]],
                             selrows, sem).wait()
            pltpu.sync_copy(selrows,
                            aggsel_o.at[c, pl.ds(s * BS + hh * H, H), :])

    return k


def _make_head_kernel(Np, D, B, N):
    nb = Np // BLK

    def body(agg_ref, rin_ref, w_ref, b_ref, g_ref, be_ref,
             aggsel_ref, rinsel_ref, embrel_ref, out_ref,
             h_scr, s_acc, ss_acc, mean_s, rstd_s, q_scr):
        p = pl.program_id(0)
        j = pl.program_id(1)

        @pl.when(p == 0)
        def _pass0():
            agg = agg_ref[0] + agg_ref[1]
            a = agg * rin_ref[...]
            h = jnp.dot(a, w_ref[...],
                        preferred_element_type=jnp.float32) + b_ref[...]
            rowid = lax.broadcasted_iota(jnp.int32, (BLK, 1), 0) + j * BLK
            hm = jnp.where(rowid < N, h, 0.0)

            @pl.when(j == 0)
            def _init():
                s_acc[...] = jnp.zeros_like(s_acc)
                ss_acc[...] = jnp.zeros_like(ss_acc)

            s_acc[...] += jnp.sum(hm, axis=0, keepdims=True)
            ss_acc[...] += jnp.sum(hm * hm, axis=0, keepdims=True)
            h_scr[pl.ds(j * BLK, BLK), :] = h

        @pl.when(p == 1)
        def _pass1():
            @pl.when(j == 0)
            def _head():
                mean = s_acc[...] * (1.0 / N)
                var = ss_acc[...] * (1.0 / N) - mean * mean
                rstd = lax.rsqrt(var + 1e-5)
                mean_s[...] = mean
                rstd_s[...] = rstd
                aggsel = aggsel_ref[0] + aggsel_ref[1]
                hsel = jnp.dot(aggsel * rinsel_ref[...], w_ref[...],
                               preferred_element_type=jnp.float32) + b_ref[...]
                embe1 = jnp.tanh((hsel - mean) * rstd * g_ref[...]
                                 + be_ref[...])
                q_scr[...] = embe1 * embrel_ref[...]

            hj = h_scr[pl.ds(j * BLK, BLK), :]
            nbk = jnp.tanh((hj - mean_s[...]) * rstd_s[...] * g_ref[...]
                           + be_ref[...])
            lg = lax.dot_general(q_scr[...], nbk,
                                 (((1,), (1,)), ((), ())),
                                 preferred_element_type=jnp.float32)
            out_ref[...] = jax.nn.sigmoid(lg)

    return pl.pallas_call(
        body,
        grid=(2, nb),
        in_specs=[
            pl.BlockSpec((NC, BLK, D), lambda p, j: (0, j * (1 - p), 0)),
            pl.BlockSpec((BLK, 1), lambda p, j: (j * (1 - p), 0)),
            pl.BlockSpec((D, D), lambda p, j: (0, 0)),
            pl.BlockSpec((1, D), lambda p, j: (0, 0)),
            pl.BlockSpec((1, D), lambda p, j: (0, 0)),
            pl.BlockSpec((1, D), lambda p, j: (0, 0)),
            pl.BlockSpec((NC, B, D), lambda p, j: (0, 0, 0)),
            pl.BlockSpec((B, 1), lambda p, j: (0, 0)),
            pl.BlockSpec((B, D), lambda p, j: (0, 0)),
        ],
        out_specs=pl.BlockSpec((B, BLK), lambda p, j: (0, p * j)),
        out_shape=jax.ShapeDtypeStruct((B, N), jnp.float32),
        scratch_shapes=[
            pltpu.VMEM((Np, D), jnp.float32),
            pltpu.VMEM((1, D), jnp.float32),
            pltpu.VMEM((1, D), jnp.float32),
            pltpu.VMEM((1, D), jnp.float32),
            pltpu.VMEM((1, D), jnp.float32),
            pltpu.VMEM((B, D), jnp.float32),
        ],
    )


def kernel(node_emb, W, b, gamma, beta, rel_emb, edge_index, e1, rel):
    N, D = node_emb.shape
    E = edge_index.shape[1]
    B = e1.shape[0]
    R = rel_emb.shape[0]
    Np = ((N + BLK - 1) // BLK) * BLK

    e3 = edge_index.reshape(2, NW, 4, E // (NW * 4 * 125), 125)

    dego_p, degi_p, emb_rel_sel = _make_deg_kernel(E, Np, B, R, D)(
        e3, rel, rel_emb)

    y, rin_col = _make_scale_kernel(N, Np, D)(
        node_emb, dego_p.reshape(NC, Np, 1), degi_p.reshape(NC, Np, 1))

    aggp, aggselp, rinsel = _make_agg_kernel(E, Np, B, D)(
        y, e3, e1, rin_col.reshape(Np))

    return _make_head_kernel(Np, D, B, N)(
        aggp, rin_col, W, b.reshape(1, D), gamma.reshape(1, D),
        beta.reshape(1, D), aggselp, rinsel.reshape(B, 1), emb_rel_sel)
